# Initial kernel scaffold; baseline (speedup 1.0000x reference)
import jax
import jax.numpy as jnp
from jax.experimental import pallas as pl


def kernel(rpn_probs, rpn_bbox, anchors):
    raise NotImplementedError("write your pallas kernel here")


# speedup vs baseline: 4.2555x; 4.2555x over previous
import jax
import jax.numpy as jnp
from jax.experimental import pallas as pl
from jax.experimental.pallas import tpu as pltpu

_B = 8
_N = 20000
_ROWS = 160
_LANES = 128
_NP = _ROWS * _LANES
_K = 6000
_OUT = 1000
_THR = 0.7


def _proposal_kernel(scores_ref, anc_ref, dlt_ref, out_ref,
                     sm_ref, y1_ref, x1_ref, y2_ref, x2_ref, ar_ref):
    a_y1 = anc_ref[0, 0]
    a_x1 = anc_ref[0, 1]
    a_y2 = anc_ref[0, 2]
    a_x2 = anc_ref[0, 3]
    d_y = dlt_ref[0, 0] * 0.1
    d_x = dlt_ref[0, 1] * 0.1
    d_h = dlt_ref[0, 2] * 0.2
    d_w = dlt_ref[0, 3] * 0.2
    h = a_y2 - a_y1
    w = a_x2 - a_x1
    cy = a_y1 + 0.5 * h + d_y * h
    cx = a_x1 + 0.5 * w + d_x * w
    hh = h * jnp.exp(d_h)
    ww = w * jnp.exp(d_w)
    ry1 = cy - 0.5 * hh
    rx1 = cx - 0.5 * ww
    y1 = jnp.clip(ry1, 0.0, 1.0)
    x1 = jnp.clip(rx1, 0.0, 1.0)
    y2 = jnp.clip(ry1 + hh, 0.0, 1.0)
    x2 = jnp.clip(rx1 + ww, 0.0, 1.0)
    y1_ref[...] = y1
    x1_ref[...] = x1
    y2_ref[...] = y2
    x2_ref[...] = x2
    ar_ref[...] = jnp.maximum(y2 - y1, 0.0) * jnp.maximum(x2 - x1, 0.0)

    s = scores_ref[0]
    bits = jax.lax.bitcast_convert_type(s, jnp.int32)
    pos = (jax.lax.broadcasted_iota(jnp.int32, (_ROWS, _LANES), 0) * _LANES
           + jax.lax.broadcasted_iota(jnp.int32, (_ROWS, _LANES), 1))

    def bs_body(_, lohi):
        lo, hi = lohi
        mid = (lo + hi) // 2
        c = jnp.sum(jnp.where(bits >= mid, 1, 0))
        big = c >= _K
        return (jnp.where(big, mid, lo), jnp.where(big, hi, mid))

    lo, hi = jax.lax.fori_loop(
        0, 31, bs_body, (jnp.int32(0), jnp.int32(0x3F800000)))
    tau = lo
    c_gt = jnp.sum(jnp.where(bits > tau, 1, 0))
    k_eq = _K - c_gt
    eqt = bits == tau

    def bs2_body(_, lohi):
        lo2, hi2 = lohi
        mid2 = (lo2 + hi2) // 2
        c2 = jnp.sum(jnp.where(eqt & (pos < mid2), 1, 0))
        big2 = c2 >= k_eq
        return (jnp.where(big2, lo2, mid2), jnp.where(big2, mid2, hi2))

    _, m_idx = jax.lax.fori_loop(
        0, 15, bs2_body, (jnp.int32(0), jnp.int32(_NP)))
    active = (bits > tau) | (eqt & (pos < m_idx))
    sm_ref[...] = jnp.where(active, s, -1.0)

    lane_iota = jax.lax.broadcasted_iota(jnp.int32, (1, _LANES), 1)

    def pick(ref, r, c):
        row = ref[pl.ds(r, 1), :]
        return jnp.sum(jnp.where(lane_iota == c, row, 0.0))

    def nms_body(i, carry):
        sm = sm_ref[...]
        mval = jnp.max(sm)
        valid = mval > 0.0
        idx = jnp.min(jnp.where(sm == mval, pos, _NP))
        r = idx // _LANES
        c = idx - r * _LANES
        by1 = pick(y1_ref, r, c)
        bx1 = pick(x1_ref, r, c)
        by2 = pick(y2_ref, r, c)
        bx2 = pick(x2_ref, r, c)
        bar = pick(ar_ref, r, c)
        iy1 = jnp.maximum(y1_ref[...], by1)
        ix1 = jnp.maximum(x1_ref[...], bx1)
        iy2 = jnp.minimum(y2_ref[...], by2)
        ix2 = jnp.minimum(x2_ref[...], bx2)
        inter = jnp.maximum(iy2 - iy1, 0.0) * jnp.maximum(ix2 - ix1, 0.0)
        union = bar + ar_ref[...] - inter
        kill = (inter > _THR * jnp.maximum(union, 1e-8)) | (pos == idx)
        sm_ref[...] = jnp.where(kill, -1.0, sm)
        zero = jnp.float32(0.0)
        out_ref[pl.ds(0, 1), pl.ds(i, 1), pl.ds(0, 1)] = (
            jnp.where(valid, by1, zero).reshape(1, 1, 1))
        out_ref[pl.ds(0, 1), pl.ds(i, 1), pl.ds(1, 1)] = (
            jnp.where(valid, bx1, zero).reshape(1, 1, 1))
        out_ref[pl.ds(0, 1), pl.ds(i, 1), pl.ds(2, 1)] = (
            jnp.where(valid, by2, zero).reshape(1, 1, 1))
        out_ref[pl.ds(0, 1), pl.ds(i, 1), pl.ds(3, 1)] = (
            jnp.where(valid, bx2, zero).reshape(1, 1, 1))
        return carry

    jax.lax.fori_loop(0, _OUT, nms_body, jnp.int32(0))


@jax.jit
def kernel(rpn_probs, rpn_bbox, anchors):
    scores = rpn_probs[..., 1]
    pad = _NP - _N
    scores_p = jnp.pad(scores, ((0, 0), (0, pad)),
                       constant_values=-1.0).reshape(_B, _ROWS, _LANES)
    anc_p = jnp.pad(anchors, ((0, 0), (0, pad), (0, 0))).transpose(
        (0, 2, 1)).reshape(_B, 4, _ROWS, _LANES)
    dlt_p = jnp.pad(rpn_bbox, ((0, 0), (0, pad), (0, 0))).transpose(
        (0, 2, 1)).reshape(_B, 4, _ROWS, _LANES)
    return pl.pallas_call(
        _proposal_kernel,
        grid=(_B,),
        in_specs=[
            pl.BlockSpec((1, _ROWS, _LANES), lambda b: (b, 0, 0)),
            pl.BlockSpec((1, 4, _ROWS, _LANES), lambda b: (b, 0, 0, 0)),
            pl.BlockSpec((1, 4, _ROWS, _LANES), lambda b: (b, 0, 0, 0)),
        ],
        out_specs=pl.BlockSpec((1, _OUT, 4), lambda b: (b, 0, 0)),
        out_shape=jax.ShapeDtypeStruct((_B, _OUT, 4), jnp.float32),
        scratch_shapes=[
            pltpu.VMEM((_ROWS, _LANES), jnp.float32),
            pltpu.VMEM((_ROWS, _LANES), jnp.float32),
            pltpu.VMEM((_ROWS, _LANES), jnp.float32),
            pltpu.VMEM((_ROWS, _LANES), jnp.float32),
            pltpu.VMEM((_ROWS, _LANES), jnp.float32),
            pltpu.VMEM((_ROWS, _LANES), jnp.float32),
        ],
    )(scores_p, anc_p, dlt_p)
